# Initial kernel scaffold; baseline (speedup 1.0000x reference)
import jax
import jax.numpy as jnp
from jax.experimental import pallas as pl


def kernel(gold_encoding_inds, latents, epc, codebook):
    raise NotImplementedError("write your pallas kernel here")


# speedup vs baseline: 1.4201x; 1.4201x over previous
import functools

import jax
import jax.numpy as jnp
from jax import lax
from jax.experimental import pallas as pl
from jax.experimental.pallas import tpu as pltpu
from jax.experimental.pallas import tpu_sc as plsc

B, T, D, K = 16, 576, 64, 8192
N = B * T
NC, NS, L = 2, 16, 16
NW = NC * NS
BPW = N // NW
CHUNK = 96
NCHUNK = BPW // CHUNK

_mesh = plsc.VectorSubcoreMesh(core_axis_name="c", subcore_axis_name="s")


@functools.partial(
    pl.kernel,
    mesh=_mesh,
    out_type=jax.ShapeDtypeStruct((N, D), jnp.float32),
    scratch_types=[
        pltpu.VMEM((BPW,), jnp.int32),
        pltpu.VMEM((BPW, D), jnp.float32),
        pltpu.SemaphoreType.DMA,
    ],
    compiler_params=pltpu.CompilerParams(use_tc_tiling_on_sc=False),
)
def _sc_gather(idx_hbm, table_hbm, q_hbm, idx_v, rows_v, sem):
    wid = lax.axis_index("s") * NC + lax.axis_index("c")
    base = wid * BPW
    pltpu.sync_copy(idx_hbm.at[pl.ds(base, BPW)], idx_v)
    copies = [
        pltpu.async_copy(
            table_hbm.at[idx_v.at[pl.ds(c * CHUNK, CHUNK)]],
            rows_v.at[pl.ds(c * CHUNK, CHUNK)],
            sem,
        )
        for c in range(NCHUNK)
    ]
    for cp in copies:
        cp.wait()
    pltpu.sync_copy(rows_v, q_hbm.at[pl.ds(base, BPW)])


def _tc_loss_body(q_ref, l_ref, o_ref):
    d = q_ref[...] - l_ref[...]
    o_ref[...] = jnp.sum(d * d, axis=-1) * (1.25 / D)


_ROWS, _COLS = 72, 128


def _tc_loss(q_flat, lat_flat):
    return pl.pallas_call(
        _tc_loss_body,
        out_shape=jax.ShapeDtypeStruct((_ROWS, _COLS), jnp.float32),
    )(q_flat.reshape(_ROWS, _COLS, D), lat_flat.reshape(_ROWS, _COLS, D))


def kernel(gold_encoding_inds, latents, epc, codebook):
    idx = gold_encoding_inds.reshape(N)
    lat_flat = latents.reshape(N, D)
    q_flat = _sc_gather(idx, codebook)
    loss = _tc_loss(q_flat, lat_flat)
    return (
        q_flat.reshape(B, T, D),
        loss.reshape(B, T),
        gold_encoding_inds.T,
    )
